# baseline (speedup 1.0000x reference)
import jax
import jax.numpy as jnp
from jax import lax
from jax.experimental import pallas as pl
from jax.experimental.pallas import tpu as pltpu

T = 4096
QUES = 2048
H = 1024
K = 64
NA = 16
BA = T // NA
NG = 24
BG = 128
KC = 8


def _body(score_ref, q_ref, h0_ref, qs_ref, wih_ref, whh_ref, bih_ref,
          bhh_ref, ws_ref, bs_ref, hs_ref, pred_ref, h_ref,
          a_scr, v_scr, g_scr, r_scr, z_scr, sem):
    s = pl.program_id(0)
    q = q_ref[...]
    h0 = h0_ref[...]

    @pl.when(s < NA)
    def _alpha():
        ab = lax.dot_general(q, qs_ref[...], (((1,), (1,)), ((), ())),
                             preferred_element_type=jnp.float32)
        a_scr[pl.ds(s, 1), :] = ab

    @pl.when((s >= NA) & (s < NA + 8))
    def _topk_chunk():
        row = lax.broadcasted_iota(jnp.int32, (NA, BA), 0)
        col = lax.broadcasted_iota(jnp.int32, (NA, BA), 1)
        pos = row * BA + col
        lane = lax.broadcasted_iota(jnp.int32, (1, K), 1)
        big = jnp.int32(2**30)
        neg = jnp.float32(-jnp.inf)
        j0 = (s - NA) * KC

        def body(i, carry):
            a, vals = carry
            m = jnp.max(a)
            fi = jnp.min(jnp.where(a == m, pos, big))
            cp = pltpu.make_async_copy(hs_ref.at[pl.ds(fi, 1), 0, :],
                                       g_scr.at[pl.ds(j0 + i, 1), :], sem)
            cp.start()
            a = jnp.where(pos == fi, neg, a)
            vals = jnp.where(lane == j0 + i, m, vals)
            return a, vals

        a, vals = lax.fori_loop(0, KC, body, (a_scr[...], v_scr[...]))
        a_scr[...] = a
        v_scr[...] = vals

    @pl.when(s == NG)
    def _attn():
        pltpu.make_async_copy(hs_ref.at[pl.ds(0, K), 0, :], g_scr, sem).wait()
        vals = v_scr[...]
        e = jnp.exp(vals - jnp.max(vals))
        w = e / jnp.sum(e)
        attn = lax.dot_general(w, g_scr[...], (((1,), (0,)), ((), ())),
                               preferred_element_type=jnp.float32)
        ws = ws_ref[...]
        pred_ref[0, 0] = (jnp.sum(ws[:, :QUES] * q)
                          + jnp.sum(ws[:, QUES:] * attn) + bs_ref[0, 0])

    @pl.when(s < NG)
    def _gru():
        t = s
        c = lax.rem(t, 8)
        off = c * BG
        flag = score_ref[0, 0] >= 0.5
        m1 = jnp.where(flag, 1.0, 0.0)
        m2 = jnp.where(flag, 0.0, 1.0)
        x = jnp.concatenate([q * m1, q * m2], axis=1)
        gi = lax.dot_general(x, wih_ref[...], (((1,), (1,)), ((), ())),
                             preferred_element_type=jnp.float32)
        gh = lax.dot_general(h0, whh_ref[...], (((1,), (1,)), ((), ())),
                             preferred_element_type=jnp.float32)

        @pl.when(t < 8)
        def _r():
            gi0 = gi + bih_ref[pl.ds(0, 1), pl.ds(off, BG)]
            gh0 = gh + bhh_ref[pl.ds(0, 1), pl.ds(off, BG)]
            r_scr[pl.ds(0, 1), pl.ds(off, BG)] = jax.nn.sigmoid(gi0 + gh0)

        @pl.when((t >= 8) & (t < 16))
        def _z():
            gi1 = gi + bih_ref[pl.ds(1, 1), pl.ds(off, BG)]
            gh1 = gh + bhh_ref[pl.ds(1, 1), pl.ds(off, BG)]
            z_scr[pl.ds(0, 1), pl.ds(off, BG)] = jax.nn.sigmoid(gi1 + gh1)

        @pl.when(t >= 16)
        def _n():
            gi2 = gi + bih_ref[pl.ds(2, 1), pl.ds(off, BG)]
            gh2 = gh + bhh_ref[pl.ds(2, 1), pl.ds(off, BG)]
            r = r_scr[pl.ds(0, 1), pl.ds(off, BG)]
            z = z_scr[pl.ds(0, 1), pl.ds(off, BG)]
            n = jnp.tanh(gi2 + r * gh2)
            h0c = h0_ref[pl.ds(0, 1), pl.ds(off, BG)]
            h_ref[pl.ds(0, 1), pl.ds(off, BG)] = (1.0 - z) * n + z * h0c


def kernel(question, score, questions, hs, initial_h, W_ih, W_hh, b_ih, b_hh,
           W_score, b_score):
    q2 = question.reshape(1, QUES)
    h0 = hs[T - 1, 0].reshape(1, H)

    pred, h_new = pl.pallas_call(
        _body,
        grid=(NG + 1,),
        in_specs=[
            pl.BlockSpec((1, 1), lambda s: (0, 0), memory_space=pltpu.SMEM),
            pl.BlockSpec((1, QUES), lambda s: (0, 0)),
            pl.BlockSpec((1, H), lambda s: (0, 0)),
            pl.BlockSpec((BA, QUES), lambda s: (jnp.minimum(s, NA - 1), 0)),
            pl.BlockSpec((BG, 2 * QUES),
                         lambda s: (jnp.minimum(s, NG - 1), 0)),
            pl.BlockSpec((BG, H),
                         lambda s: (jnp.minimum(s, NG - 1), 0)),
            pl.BlockSpec((3, H), lambda s: (0, 0)),
            pl.BlockSpec((3, H), lambda s: (0, 0)),
            pl.BlockSpec((1, QUES + H), lambda s: (0, 0)),
            pl.BlockSpec((1, 1), lambda s: (0, 0), memory_space=pltpu.SMEM),
            pl.BlockSpec(memory_space=pl.ANY),
        ],
        out_specs=[
            pl.BlockSpec((1, 1), lambda s: (0, 0), memory_space=pltpu.SMEM),
            pl.BlockSpec((1, H), lambda s: (0, 0)),
        ],
        out_shape=[
            jax.ShapeDtypeStruct((1, 1), jnp.float32),
            jax.ShapeDtypeStruct((1, H), jnp.float32),
        ],
        scratch_shapes=[
            pltpu.VMEM((NA, BA), jnp.float32),
            pltpu.VMEM((1, K), jnp.float32),
            pltpu.VMEM((K, H), jnp.float32),
            pltpu.VMEM((1, H), jnp.float32),
            pltpu.VMEM((1, H), jnp.float32),
            pltpu.SemaphoreType.DMA,
        ],
    )(score.reshape(1, 1), q2, h0, questions, W_ih, W_hh, b_ih.reshape(3, H),
      b_hh.reshape(3, H), W_score, b_score.reshape(1, 1), hs)

    return pred.reshape(1), h_new.reshape(1, 1, H)


# speedup vs baseline: 1.0344x; 1.0344x over previous
import jax
import jax.numpy as jnp
from jax import lax
from jax.experimental import pallas as pl
from jax.experimental.pallas import tpu as pltpu

T = 4096
QUES = 2048
H = 1024
K = 64
NA = 16
BA = T // NA
NG = 6
BG = 512
KC = 16


def _body(score_ref, q_ref, h0_ref, qs_ref, wih_ref, whh_ref, bih_ref,
          bhh_ref, ws_ref, bs_ref, hs_ref, pred_ref, h_ref,
          a_scr, v_scr, g_scr, r_scr, z_scr, sem):
    s = pl.program_id(0)
    q = q_ref[...]
    h0 = h0_ref[...]

    @pl.when(s < NA)
    def _alpha():
        ab = lax.dot_general(q, qs_ref[...], (((1,), (1,)), ((), ())),
                             preferred_element_type=jnp.float32)
        a_scr[pl.ds(s, 1), :] = ab

    @pl.when((s >= NA) & (s < NA + 4))
    def _topk_chunk():
        row = lax.broadcasted_iota(jnp.int32, (NA, BA), 0)
        col = lax.broadcasted_iota(jnp.int32, (NA, BA), 1)
        pos = row * BA + col
        lane = lax.broadcasted_iota(jnp.int32, (1, K), 1)
        big = jnp.int32(2**30)
        neg = jnp.float32(-jnp.inf)
        j0 = (s - NA) * KC

        def body(i, carry):
            a, vals = carry
            m = jnp.max(a)
            fi = jnp.min(jnp.where(a == m, pos, big))
            cp = pltpu.make_async_copy(hs_ref.at[pl.ds(fi, 1), 0, :],
                                       g_scr.at[pl.ds(j0 + i, 1), :], sem)
            cp.start()
            a = jnp.where(pos == fi, neg, a)
            vals = jnp.where(lane == j0 + i, m, vals)
            return a, vals

        a, vals = lax.fori_loop(0, KC, body, (a_scr[...], v_scr[...]))
        a_scr[...] = a
        v_scr[...] = vals

    @pl.when(s == NA + 4)
    def _attn():
        pltpu.make_async_copy(hs_ref.at[pl.ds(0, K), 0, :], g_scr, sem).wait()
        vals = v_scr[...]
        e = jnp.exp(vals - jnp.max(vals))
        w = e / jnp.sum(e)
        attn = lax.dot_general(w, g_scr[...], (((1,), (0,)), ((), ())),
                               preferred_element_type=jnp.float32)
        ws = ws_ref[...]
        pred_ref[0, 0] = (jnp.sum(ws[:, :QUES] * q)
                          + jnp.sum(ws[:, QUES:] * attn) + bs_ref[0, 0])

    @pl.when(s >= NA)
    def _gru():
        t = s - NA
        c = lax.rem(t, 2)
        off = c * BG
        flag = score_ref[0, 0] >= 0.5
        m1 = jnp.where(flag, 1.0, 0.0)
        m2 = jnp.where(flag, 0.0, 1.0)
        x = jnp.concatenate([q * m1, q * m2], axis=1)
        gi = lax.dot_general(x, wih_ref[...], (((1,), (1,)), ((), ())),
                             preferred_element_type=jnp.float32)
        gh = lax.dot_general(h0, whh_ref[...], (((1,), (1,)), ((), ())),
                             preferred_element_type=jnp.float32)

        @pl.when(t < 2)
        def _r():
            gi0 = gi + bih_ref[pl.ds(0, 1), pl.ds(off, BG)]
            gh0 = gh + bhh_ref[pl.ds(0, 1), pl.ds(off, BG)]
            r_scr[pl.ds(0, 1), pl.ds(off, BG)] = jax.nn.sigmoid(gi0 + gh0)

        @pl.when((t >= 2) & (t < 4))
        def _z():
            gi1 = gi + bih_ref[pl.ds(1, 1), pl.ds(off, BG)]
            gh1 = gh + bhh_ref[pl.ds(1, 1), pl.ds(off, BG)]
            z_scr[pl.ds(0, 1), pl.ds(off, BG)] = jax.nn.sigmoid(gi1 + gh1)

        @pl.when(t >= 4)
        def _n():
            gi2 = gi + bih_ref[pl.ds(2, 1), pl.ds(off, BG)]
            gh2 = gh + bhh_ref[pl.ds(2, 1), pl.ds(off, BG)]
            r = r_scr[pl.ds(0, 1), pl.ds(off, BG)]
            z = z_scr[pl.ds(0, 1), pl.ds(off, BG)]
            n = jnp.tanh(gi2 + r * gh2)
            h0c = h0_ref[pl.ds(0, 1), pl.ds(off, BG)]
            h_ref[pl.ds(0, 1), pl.ds(off, BG)] = (1.0 - z) * n + z * h0c


def kernel(question, score, questions, hs, initial_h, W_ih, W_hh, b_ih, b_hh,
           W_score, b_score):
    q2 = question.reshape(1, QUES)
    h0 = hs[T - 1, 0].reshape(1, H)

    pred, h_new = pl.pallas_call(
        _body,
        grid=(NA + NG,),
        in_specs=[
            pl.BlockSpec((1, 1), lambda s: (0, 0), memory_space=pltpu.SMEM),
            pl.BlockSpec((1, QUES), lambda s: (0, 0)),
            pl.BlockSpec((1, H), lambda s: (0, 0)),
            pl.BlockSpec((BA, QUES), lambda s: (jnp.minimum(s, NA - 1), 0)),
            pl.BlockSpec((BG, 2 * QUES),
                         lambda s: (jnp.clip(s - NA, 0, NG - 1), 0)),
            pl.BlockSpec((BG, H),
                         lambda s: (jnp.clip(s - NA, 0, NG - 1), 0)),
            pl.BlockSpec((3, H), lambda s: (0, 0)),
            pl.BlockSpec((3, H), lambda s: (0, 0)),
            pl.BlockSpec((1, QUES + H), lambda s: (0, 0)),
            pl.BlockSpec((1, 1), lambda s: (0, 0), memory_space=pltpu.SMEM),
            pl.BlockSpec(memory_space=pl.ANY),
        ],
        out_specs=[
            pl.BlockSpec((1, 1), lambda s: (0, 0), memory_space=pltpu.SMEM),
            pl.BlockSpec((1, H), lambda s: (0, 0)),
        ],
        out_shape=[
            jax.ShapeDtypeStruct((1, 1), jnp.float32),
            jax.ShapeDtypeStruct((1, H), jnp.float32),
        ],
        scratch_shapes=[
            pltpu.VMEM((NA, BA), jnp.float32),
            pltpu.VMEM((1, K), jnp.float32),
            pltpu.VMEM((K, H), jnp.float32),
            pltpu.VMEM((1, H), jnp.float32),
            pltpu.VMEM((1, H), jnp.float32),
            pltpu.SemaphoreType.DMA,
        ],
    )(score.reshape(1, 1), q2, h0, questions, W_ih, W_hh, b_ih.reshape(3, H),
      b_hh.reshape(3, H), W_score, b_score.reshape(1, 1), hs)

    return pred.reshape(1), h_new.reshape(1, 1, H)


# speedup vs baseline: 1.1047x; 1.0680x over previous
import jax
import jax.numpy as jnp
from jax import lax
from jax.experimental import pallas as pl
from jax.experimental.pallas import tpu as pltpu

T = 4096
QUES = 2048
H = 1024
K = 64
NA = 8
BA = T // NA
NG = 6
BG = 512
KC = 16


def _body(score_ref, q_ref, h0_ref, qs_ref, wih_ref, whh_ref, bih_ref,
          bhh_ref, ws_ref, bs_ref, hs_ref, pred_ref, h_ref,
          a_scr, v_scr, g_scr, r_scr, z_scr, sem):
    s = pl.program_id(0)
    q = q_ref[...]
    h0 = h0_ref[...]

    @pl.when(s < NA)
    def _alpha():
        ab = lax.dot_general(q, qs_ref[...], (((1,), (1,)), ((), ())),
                             preferred_element_type=jnp.float32)
        a_scr[pl.ds(s, 1), :] = ab

    @pl.when((s >= NA) & (s < NA + 4))
    def _topk_chunk():
        row = lax.broadcasted_iota(jnp.int32, (NA, BA), 0)
        col = lax.broadcasted_iota(jnp.int32, (NA, BA), 1)
        pos = row * BA + col
        lane = lax.broadcasted_iota(jnp.int32, (1, K), 1)
        big = jnp.int32(2**30)
        neg = jnp.float32(-jnp.inf)
        j0 = (s - NA) * KC

        def body(i, carry):
            a, vals = carry
            m = jnp.max(a)
            fi = jnp.min(jnp.where(a == m, pos, big))
            cp = pltpu.make_async_copy(hs_ref.at[pl.ds(fi, 1), 0, :],
                                       g_scr.at[pl.ds(j0 + i, 1), :], sem)
            cp.start()
            a = jnp.where(pos == fi, neg, a)
            vals = jnp.where(lane == j0 + i, m, vals)
            return a, vals

        a, vals = lax.fori_loop(0, KC, body, (a_scr[...], v_scr[...]))
        a_scr[...] = a
        v_scr[...] = vals

    @pl.when(s == NA + 4)
    def _attn():
        pltpu.make_async_copy(hs_ref.at[pl.ds(0, K), 0, :], g_scr, sem).wait()
        vals = v_scr[...]
        e = jnp.exp(vals - jnp.max(vals))
        w = e / jnp.sum(e)
        attn = lax.dot_general(w, g_scr[...], (((1,), (0,)), ((), ())),
                               preferred_element_type=jnp.float32)
        ws = ws_ref[...]
        pred_ref[0, 0] = (jnp.sum(ws[:, :QUES] * q)
                          + jnp.sum(ws[:, QUES:] * attn) + bs_ref[0, 0])

    @pl.when(s >= NA)
    def _gru():
        t = s - NA
        c = lax.rem(t, 2)
        off = c * BG
        flag = score_ref[0, 0] >= 0.5
        m1 = jnp.where(flag, 1.0, 0.0)
        m2 = jnp.where(flag, 0.0, 1.0)
        x = jnp.concatenate([q * m1, q * m2], axis=1)
        gi = lax.dot_general(x, wih_ref[...], (((1,), (1,)), ((), ())),
                             preferred_element_type=jnp.float32)
        gh = lax.dot_general(h0, whh_ref[...], (((1,), (1,)), ((), ())),
                             preferred_element_type=jnp.float32)

        @pl.when(t < 2)
        def _r():
            gi0 = gi + bih_ref[pl.ds(0, 1), pl.ds(off, BG)]
            gh0 = gh + bhh_ref[pl.ds(0, 1), pl.ds(off, BG)]
            r_scr[pl.ds(0, 1), pl.ds(off, BG)] = jax.nn.sigmoid(gi0 + gh0)

        @pl.when((t >= 2) & (t < 4))
        def _z():
            gi1 = gi + bih_ref[pl.ds(1, 1), pl.ds(off, BG)]
            gh1 = gh + bhh_ref[pl.ds(1, 1), pl.ds(off, BG)]
            z_scr[pl.ds(0, 1), pl.ds(off, BG)] = jax.nn.sigmoid(gi1 + gh1)

        @pl.when(t >= 4)
        def _n():
            gi2 = gi + bih_ref[pl.ds(2, 1), pl.ds(off, BG)]
            gh2 = gh + bhh_ref[pl.ds(2, 1), pl.ds(off, BG)]
            r = r_scr[pl.ds(0, 1), pl.ds(off, BG)]
            z = z_scr[pl.ds(0, 1), pl.ds(off, BG)]
            n = jnp.tanh(gi2 + r * gh2)
            h0c = h0_ref[pl.ds(0, 1), pl.ds(off, BG)]
            h_ref[pl.ds(0, 1), pl.ds(off, BG)] = (1.0 - z) * n + z * h0c


def kernel(question, score, questions, hs, initial_h, W_ih, W_hh, b_ih, b_hh,
           W_score, b_score):
    q2 = question.reshape(1, QUES)
    h0 = hs[T - 1, 0].reshape(1, H)

    pred, h_new = pl.pallas_call(
        _body,
        grid=(NA + NG,),
        in_specs=[
            pl.BlockSpec((1, 1), lambda s: (0, 0), memory_space=pltpu.SMEM),
            pl.BlockSpec((1, QUES), lambda s: (0, 0)),
            pl.BlockSpec((1, H), lambda s: (0, 0)),
            pl.BlockSpec((BA, QUES), lambda s: (jnp.minimum(s, NA - 1), 0)),
            pl.BlockSpec((BG, 2 * QUES),
                         lambda s: (jnp.clip(s - NA, 0, NG - 1), 0)),
            pl.BlockSpec((BG, H),
                         lambda s: (jnp.clip(s - NA, 0, NG - 1), 0)),
            pl.BlockSpec((3, H), lambda s: (0, 0)),
            pl.BlockSpec((3, H), lambda s: (0, 0)),
            pl.BlockSpec((1, QUES + H), lambda s: (0, 0)),
            pl.BlockSpec((1, 1), lambda s: (0, 0), memory_space=pltpu.SMEM),
            pl.BlockSpec(memory_space=pl.ANY),
        ],
        out_specs=[
            pl.BlockSpec((1, 1), lambda s: (0, 0), memory_space=pltpu.SMEM),
            pl.BlockSpec((1, H), lambda s: (0, 0)),
        ],
        out_shape=[
            jax.ShapeDtypeStruct((1, 1), jnp.float32),
            jax.ShapeDtypeStruct((1, H), jnp.float32),
        ],
        scratch_shapes=[
            pltpu.VMEM((NA, BA), jnp.float32),
            pltpu.VMEM((1, K), jnp.float32),
            pltpu.VMEM((K, H), jnp.float32),
            pltpu.VMEM((1, H), jnp.float32),
            pltpu.VMEM((1, H), jnp.float32),
            pltpu.SemaphoreType.DMA,
        ],
    )(score.reshape(1, 1), q2, h0, questions, W_ih, W_hh, b_ih.reshape(3, H),
      b_hh.reshape(3, H), W_score, b_score.reshape(1, 1), hs)

    return pred.reshape(1), h_new.reshape(1, 1, H)
